# Initial kernel scaffold; baseline (speedup 1.0000x reference)
import jax
import jax.numpy as jnp
from jax.experimental import pallas as pl


def kernel(x, edge_index, batch, bn0_g, bn0_b, W0, b0, bn1_g, bn1_b, W1, b1, l1W, l1b, l2W, l2b):
    raise NotImplementedError("write your pallas kernel here")


# speedup vs baseline: 13.0595x; 13.0595x over previous
import functools

import jax
import jax.numpy as jnp
from jax import lax
from jax.experimental import pallas as pl
from jax.experimental.pallas import tpu as pltpu
from jax.experimental.pallas import tpu_sc as plsc

N = 10000
NPAD = 10240
E = 320000
NS = 16
NC = 2
K = 125
RPT = NPAD // NS

_HIGH = jax.lax.Precision.HIGHEST
DH = 128
CB = 16


def _dot(a, b):
    return jax.lax.dot_general(a, b, (((1,), (0,)), ((), ())),
                               precision=_HIGH,
                               preferred_element_type=jnp.float32)


def _make_deg_kernel():
    nch = E // (NC * NS) // K
    mesh = plsc.VectorSubcoreMesh(core_axis_name="c", subcore_axis_name="s")

    @functools.partial(
        pl.kernel,
        out_type=jax.ShapeDtypeStruct((NC, NPAD, DH), jnp.float32),
        mesh=mesh,
        scratch_types=[
            pltpu.VMEM((nch, K), jnp.int32),
            pltpu.VMEM((K, DH), jnp.float32),
            pltpu.VMEM_SHARED((NPAD, DH), jnp.float32),
        ],
    )
    def deg_kernel(dst_hbm, zeros_hbm, ones_hbm, out_hbm, dst_v, ones_v, deg_sh):
        c = lax.axis_index("c")
        s = lax.axis_index("s")
        w = c * NS + s
        pltpu.sync_copy(dst_hbm.at[w], dst_v)
        pltpu.sync_copy(ones_hbm, ones_v)
        pltpu.sync_copy(zeros_hbm, deg_sh.at[pl.ds(s * RPT, RPT)])
        plsc.subcore_barrier()

        def body(j, carry):
            pltpu.sync_copy(ones_v, deg_sh.at[dst_v.at[j]], add=True)
            return carry

        lax.fori_loop(0, nch, body, 0)
        plsc.subcore_barrier()
        pltpu.sync_copy(deg_sh.at[pl.ds(s * RPT, RPT)],
                        out_hbm.at[c, pl.ds(s * RPT, RPT)])

    return deg_kernel


def _make_agg_kernel():
    nch = E // NS // K
    mesh = plsc.VectorSubcoreMesh(core_axis_name="c", subcore_axis_name="s")

    @functools.partial(
        pl.kernel,
        out_type=jax.ShapeDtypeStruct((NC, NPAD, DH), jnp.float32),
        mesh=mesh,
        scratch_types=[
            pltpu.VMEM((CB, K), jnp.int32),
            pltpu.VMEM((CB, K), jnp.int32),
            pltpu.VMEM((K, DH), jnp.float32),
            pltpu.VMEM_SHARED((NPAD, DH), jnp.float32),
            pltpu.SemaphoreType.DMA,
        ],
    )
    def agg_kernel(ys_hbm, srcg_hbm, dst_hbm, zeros_hbm, out_hbm,
                   src_v, dst_v, rows_v, acc_sh, sem):
        c = lax.axis_index("c")
        s = lax.axis_index("s")
        w = c * NS + s
        pltpu.sync_copy(zeros_hbm, acc_sh.at[pl.ds(s * RPT, RPT)])
        plsc.subcore_barrier()

        def outer(b, carry):
            pltpu.sync_copy(srcg_hbm.at[w, pl.ds(b * CB, CB)], src_v)
            pltpu.sync_copy(dst_hbm.at[s, pl.ds(b * CB, CB)], dst_v)

            def body(j, carry2):
                pltpu.async_copy(ys_hbm.at[src_v.at[j]], rows_v, sem).wait()
                pltpu.sync_copy(rows_v, acc_sh.at[dst_v.at[j]], add=True)
                return carry2

            lax.fori_loop(0, CB, body, 0)
            return carry

        lax.fori_loop(0, nch // CB, outer, 0)
        plsc.subcore_barrier()
        pltpu.sync_copy(acc_sh.at[pl.ds(s * RPT, RPT)],
                        out_hbm.at[c, pl.ds(s * RPT, RPT)])

    return agg_kernel


NBLK = 5
BLK = N // NBLK


def _stats_body(x_ref, out_ref):
    i = pl.program_id(0)
    xv = x_ref[...]
    s = jnp.sum(xv, axis=0, keepdims=True)
    s2 = jnp.sum(xv * xv, axis=0, keepdims=True)
    contrib = jnp.concatenate([s, s2], axis=0)

    @pl.when(i == 0)
    def _():
        out_ref[...] = contrib

    @pl.when(i > 0)
    def _():
        out_ref[...] += contrib


def _tc_stats(x, d):
    return pl.pallas_call(
        _stats_body,
        grid=(NBLK,),
        in_specs=[pl.BlockSpec((BLK, d), lambda i: (i, 0))],
        out_specs=pl.BlockSpec((2, d), lambda i: (0, 0)),
        out_shape=jax.ShapeDtypeStruct((2, d), jnp.float32),
    )(x)


def _bn_fold(stats_ref, g_ref, b_ref, w_ref):
    mean = stats_ref[0:1, :] / N
    var = stats_ref[1:2, :] / N - mean * mean
    scale = jax.lax.rsqrt(var + 1e-5) * g_ref[...]
    wp = w_ref[...] * scale.T
    bias = _dot(b_ref[...] - mean * scale, w_ref[...])
    return wp, bias


def _tc_a_body(x_ref, deg_ref, stats_ref, g_ref, b_ref, w_ref,
               ys_ref, dis_ref):
    wp, bias = _bn_fold(stats_ref, g_ref, b_ref, w_ref)
    deg = deg_ref[0, :, 0:1] + deg_ref[1, :, 0:1] + 1.0
    dis = jax.lax.rsqrt(deg)
    y = (_dot(x_ref[...], wp) + bias) * dis
    pad = jnp.zeros((BLK, DH - 96), jnp.float32)
    ys_ref[0] = jnp.concatenate([y[:, :96], pad], axis=1)
    ys_ref[1] = jnp.concatenate([y[:, 96:], pad], axis=1)
    dis_ref[...] = dis


def _finish_layer(acc_ref, ys_ref, dis_ref, bias_ref, half):
    acc = jnp.concatenate([acc_ref[0, :, :half], acc_ref[1, :, :half]], axis=1)
    y = jnp.concatenate([ys_ref[0, :, :half], ys_ref[1, :, :half]], axis=1)
    return jnp.maximum(dis_ref[...] * (acc + y) + bias_ref[...], 0.0)


def _tc_b1_body(acc_ref, ys_ref, dis_ref, b0_ref, h_ref, stats_ref):
    i = pl.program_id(0)
    h = _finish_layer(acc_ref, ys_ref, dis_ref, b0_ref, 96)
    h_ref[...] = h
    s = jnp.sum(h, axis=0, keepdims=True)
    s2 = jnp.sum(h * h, axis=0, keepdims=True)
    contrib = jnp.concatenate([s, s2], axis=0)

    @pl.when(i == 0)
    def _():
        stats_ref[...] = contrib

    @pl.when(i > 0)
    def _():
        stats_ref[...] += contrib


def _tc_b2_body(h_ref, dis_ref, stats_ref, g_ref, b_ref, w_ref, ys_ref):
    wp, bias = _bn_fold(stats_ref, g_ref, b_ref, w_ref)
    y1 = (_dot(h_ref[...], wp) + bias) * dis_ref[...]
    ys_ref[0] = y1[:, :DH]
    ys_ref[1] = y1[:, DH:]


def _tc_c_body(acc_ref, ys_ref, dis_ref, b1_ref, batch_ref,
               l1w_ref, l1b_ref, l2w_ref, l2b_ref, pool_ref, out_ref):
    i = pl.program_id(0)
    h = _finish_layer(acc_ref, ys_ref, dis_ref, b1_ref, DH)
    gids = jax.lax.broadcasted_iota(jnp.int32, (BLK, 16), 1)
    onehot = (batch_ref[...] == gids).astype(jnp.float32)
    hext = jnp.concatenate([h, jnp.ones((BLK, 1), jnp.float32)], axis=1)
    contrib = jax.lax.dot_general(onehot, hext, (((0,), (0,)), ((), ())),
                                  precision=_HIGH,
                                  preferred_element_type=jnp.float32)

    @pl.when(i == 0)
    def _():
        pool_ref[...] = contrib

    @pl.when(i > 0)
    def _():
        pool_ref[...] += contrib

    @pl.when(i == NBLK - 1)
    def _():
        pooled = pool_ref[...]
        p = pooled[:, :256] / jnp.maximum(pooled[:, 256:257], 1.0)
        o = _dot(p, l1w_ref[...]) + l1b_ref[...]
        o = _dot(o, l2w_ref[...]) + l2b_ref[...]
        out_ref[...] = o


def kernel(x, edge_index, batch, bn0_g, bn0_b, W0, b0, bn1_g, bn1_b, W1, b1,
           l1W, l1b, l2W, l2b):
    src = edge_index[0]
    dst = edge_index[1]

    nch = E // NS // K
    src_r = src.reshape(NS, nch, K)
    srcg = jnp.concatenate([src_r, src_r + N], axis=0)
    dst_r = dst.reshape(NS, nch, K)
    dst_deg = dst.reshape(NC * NS, E // (NC * NS) // K, K)

    zeros_row = jnp.zeros((RPT, DH), jnp.float32)
    ones_row = jnp.ones((K, DH), jnp.float32)

    deg2 = _make_deg_kernel()(dst_deg, zeros_row, ones_row)

    d0, d1, d2 = 128, 192, 256
    row = lambda i: (i, 0)
    stk = lambda i: (0, i, 0)
    rep2 = pl.BlockSpec((2, d0), lambda i: (0, 0))
    col = pl.BlockSpec((BLK, 1), row)

    stats0 = _tc_stats(x, d0)
    ys0, dis = pl.pallas_call(
        _tc_a_body,
        grid=(NBLK,),
        in_specs=[pl.BlockSpec((BLK, d0), row),
                  pl.BlockSpec((2, BLK, DH), stk),
                  rep2,
                  pl.BlockSpec((1, d0), lambda i: (0, 0)),
                  pl.BlockSpec((1, d0), lambda i: (0, 0)),
                  pl.BlockSpec((d0, d1), lambda i: (0, 0))],
        out_specs=(pl.BlockSpec((2, BLK, DH), stk), col),
        out_shape=(jax.ShapeDtypeStruct((2, N, DH), jnp.float32),
                   jax.ShapeDtypeStruct((N, 1), jnp.float32)),
    )(x, deg2, stats0, bn0_g.reshape(1, d0), bn0_b.reshape(1, d0), W0)

    acc0 = _make_agg_kernel()(ys0.reshape(2 * N, DH), srcg, dst_r, zeros_row)

    h0, stats1 = pl.pallas_call(
        _tc_b1_body,
        grid=(NBLK,),
        in_specs=[pl.BlockSpec((2, BLK, DH), stk),
                  pl.BlockSpec((2, BLK, DH), stk),
                  col,
                  pl.BlockSpec((1, d1), lambda i: (0, 0))],
        out_specs=(pl.BlockSpec((BLK, d1), row),
                   pl.BlockSpec((2, d1), lambda i: (0, 0))),
        out_shape=(jax.ShapeDtypeStruct((N, d1), jnp.float32),
                   jax.ShapeDtypeStruct((2, d1), jnp.float32)),
    )(acc0, ys0, dis, b0.reshape(1, d1))

    ys1 = pl.pallas_call(
        _tc_b2_body,
        grid=(NBLK,),
        in_specs=[pl.BlockSpec((BLK, d1), row),
                  col,
                  pl.BlockSpec((2, d1), lambda i: (0, 0)),
                  pl.BlockSpec((1, d1), lambda i: (0, 0)),
                  pl.BlockSpec((1, d1), lambda i: (0, 0)),
                  pl.BlockSpec((d1, d2), lambda i: (0, 0))],
        out_specs=pl.BlockSpec((2, BLK, DH), stk),
        out_shape=jax.ShapeDtypeStruct((2, N, DH), jnp.float32),
    )(h0, dis, stats1, bn1_g.reshape(1, d1), bn1_b.reshape(1, d1), W1)

    acc1 = _make_agg_kernel()(ys1.reshape(2 * N, DH), srcg, dst_r, zeros_row)

    _, out = pl.pallas_call(
        _tc_c_body,
        grid=(NBLK,),
        in_specs=[pl.BlockSpec((2, BLK, DH), stk),
                  pl.BlockSpec((2, BLK, DH), stk),
                  col,
                  pl.BlockSpec((1, d2), lambda i: (0, 0)),
                  pl.BlockSpec((BLK, 1), row),
                  pl.BlockSpec((d2, d2 // 4), lambda i: (0, 0)),
                  pl.BlockSpec((1, d2 // 4), lambda i: (0, 0)),
                  pl.BlockSpec((d2 // 4, 10), lambda i: (0, 0)),
                  pl.BlockSpec((1, 10), lambda i: (0, 0))],
        out_specs=(pl.BlockSpec((16, d2 + 1), lambda i: (0, 0)),
                   pl.BlockSpec((16, 10), lambda i: (0, 0))),
        out_shape=(jax.ShapeDtypeStruct((16, d2 + 1), jnp.float32),
                   jax.ShapeDtypeStruct((16, 10), jnp.float32)),
    )(acc1, ys1, dis, b1.reshape(1, d2), batch.reshape(N, 1),
      l1W, l1b.reshape(1, d2 // 4), l2W, l2b.reshape(1, 10))

    return out


# speedup vs baseline: 16.8592x; 1.2910x over previous
import functools

import jax
import jax.numpy as jnp
from jax import lax
from jax.experimental import pallas as pl
from jax.experimental.pallas import tpu as pltpu
from jax.experimental.pallas import tpu_sc as plsc

N = 10000
NPAD = 10240
E = 320000
NS = 16
NC = 2
K = 125
RPT = NPAD // NS

_HIGH = jax.lax.Precision.HIGHEST
DH = 128
CB = 16


def _dot(a, b):
    return jax.lax.dot_general(a, b, (((1,), (0,)), ((), ())),
                               precision=_HIGH,
                               preferred_element_type=jnp.float32)


def _make_deg_kernel():
    nch = E // (NC * NS) // K
    mesh = plsc.VectorSubcoreMesh(core_axis_name="c", subcore_axis_name="s")

    @functools.partial(
        pl.kernel,
        out_type=jax.ShapeDtypeStruct((NC, NPAD, DH), jnp.float32),
        mesh=mesh,
        scratch_types=[
            pltpu.VMEM((nch, K), jnp.int32),
            pltpu.VMEM((K, DH), jnp.float32),
            pltpu.VMEM_SHARED((NPAD, DH), jnp.float32),
        ],
    )
    def deg_kernel(dst_hbm, zeros_hbm, ones_hbm, out_hbm, dst_v, ones_v, deg_sh):
        c = lax.axis_index("c")
        s = lax.axis_index("s")
        w = c * NS + s
        pltpu.sync_copy(dst_hbm.at[w], dst_v)
        pltpu.sync_copy(ones_hbm, ones_v)
        pltpu.sync_copy(zeros_hbm, deg_sh.at[pl.ds(s * RPT, RPT)])
        plsc.subcore_barrier()

        def body(j, carry):
            pltpu.sync_copy(ones_v, deg_sh.at[dst_v.at[j]], add=True)
            return carry

        lax.fori_loop(0, nch, body, 0)
        plsc.subcore_barrier()
        pltpu.sync_copy(deg_sh.at[pl.ds(s * RPT, RPT)],
                        out_hbm.at[c, pl.ds(s * RPT, RPT)])

    return deg_kernel


def _make_agg_kernel():
    nch = E // NS // K
    mesh = plsc.VectorSubcoreMesh(core_axis_name="c", subcore_axis_name="s")

    nb = nch // CB

    @functools.partial(
        pl.kernel,
        out_type=jax.ShapeDtypeStruct((NC, NPAD, DH), jnp.float32),
        mesh=mesh,
        scratch_types=[
            pltpu.VMEM((2, CB, K), jnp.int32),
            pltpu.VMEM((2, CB, K), jnp.int32),
            pltpu.VMEM((2, K, DH), jnp.float32),
            pltpu.VMEM_SHARED((NPAD, DH), jnp.float32),
            pltpu.SemaphoreType.DMA,
            pltpu.SemaphoreType.DMA,
            pltpu.SemaphoreType.DMA,
            pltpu.SemaphoreType.DMA,
            pltpu.SemaphoreType.DMA,
        ],
    )
    def agg_kernel(ys_hbm, srcg_hbm, dst_hbm, zeros_hbm, out_hbm,
                   src_v, dst_v, rows_v, acc_sh, gs0, gs1, ss0, ss1, isem):
        c = lax.axis_index("c")
        s = lax.axis_index("s")
        w = c * NS + s
        pltpu.sync_copy(srcg_hbm.at[w, pl.ds(0, CB)], src_v.at[0])
        pltpu.sync_copy(dst_hbm.at[s, pl.ds(0, CB)], dst_v.at[0])
        pltpu.sync_copy(zeros_hbm, acc_sh.at[pl.ds(s * RPT, RPT)])
        plsc.subcore_barrier()
        gsem = (gs0, gs1)
        ssem = (ss0, ss1)

        def outer(b, carry):
            bb = b % 2

            @pl.when(b + 1 < nb)
            def _():
                pltpu.async_copy(srcg_hbm.at[w, pl.ds((b + 1) * CB, CB)],
                                 src_v.at[1 - bb], isem)
                pltpu.async_copy(dst_hbm.at[s, pl.ds((b + 1) * CB, CB)],
                                 dst_v.at[1 - bb], isem)

            dg = [None, None]
            dsc = [None, None]
            dg[0] = pltpu.async_copy(ys_hbm.at[src_v.at[bb, 0]],
                                     rows_v.at[0], gsem[0])
            for jj in range(CB):
                rb = jj % 2
                dg[rb].wait()
                if jj + 1 < CB:
                    if dsc[1 - rb] is not None:
                        dsc[1 - rb].wait()
                    dg[1 - rb] = pltpu.async_copy(
                        ys_hbm.at[src_v.at[bb, jj + 1]],
                        rows_v.at[1 - rb], gsem[1 - rb])
                dsc[rb] = pltpu.async_copy(
                    rows_v.at[rb], acc_sh.at[dst_v.at[bb, jj]],
                    ssem[rb], add=True)
            dsc[0].wait()
            dsc[1].wait()

            @pl.when(b + 1 < nb)
            def _():
                pltpu.make_async_copy(srcg_hbm.at[w, pl.ds((b + 1) * CB, CB)],
                                      src_v.at[1 - bb], isem).wait()
                pltpu.make_async_copy(dst_hbm.at[s, pl.ds((b + 1) * CB, CB)],
                                      dst_v.at[1 - bb], isem).wait()

            return carry

        lax.fori_loop(0, nb, outer, 0)
        plsc.subcore_barrier()
        pltpu.sync_copy(acc_sh.at[pl.ds(s * RPT, RPT)],
                        out_hbm.at[c, pl.ds(s * RPT, RPT)])

    return agg_kernel


NBLK = 5
BLK = N // NBLK


def _stats_body(x_ref, out_ref):
    i = pl.program_id(0)
    xv = x_ref[...]
    s = jnp.sum(xv, axis=0, keepdims=True)
    s2 = jnp.sum(xv * xv, axis=0, keepdims=True)
    contrib = jnp.concatenate([s, s2], axis=0)

    @pl.when(i == 0)
    def _():
        out_ref[...] = contrib

    @pl.when(i > 0)
    def _():
        out_ref[...] += contrib


def _tc_stats(x, d):
    return pl.pallas_call(
        _stats_body,
        grid=(NBLK,),
        in_specs=[pl.BlockSpec((BLK, d), lambda i: (i, 0))],
        out_specs=pl.BlockSpec((2, d), lambda i: (0, 0)),
        out_shape=jax.ShapeDtypeStruct((2, d), jnp.float32),
    )(x)


def _bn_fold(stats_ref, g_ref, b_ref, w_ref):
    mean = stats_ref[0:1, :] / N
    var = stats_ref[1:2, :] / N - mean * mean
    scale = jax.lax.rsqrt(var + 1e-5) * g_ref[...]
    wp = w_ref[...] * scale.T
    bias = _dot(b_ref[...] - mean * scale, w_ref[...])
    return wp, bias


def _tc_a_body(x_ref, deg_ref, stats_ref, g_ref, b_ref, w_ref,
               ys_ref, dis_ref):
    wp, bias = _bn_fold(stats_ref, g_ref, b_ref, w_ref)
    deg = deg_ref[0, :, 0:1] + deg_ref[1, :, 0:1] + 1.0
    dis = jax.lax.rsqrt(deg)
    y = (_dot(x_ref[...], wp) + bias) * dis
    pad = jnp.zeros((BLK, DH - 96), jnp.float32)
    ys_ref[0] = jnp.concatenate([y[:, :96], pad], axis=1)
    ys_ref[1] = jnp.concatenate([y[:, 96:], pad], axis=1)
    dis_ref[...] = dis


def _finish_layer(acc_ref, ys_ref, dis_ref, bias_ref, half):
    acc = jnp.concatenate([acc_ref[0, :, :half], acc_ref[1, :, :half]], axis=1)
    y = jnp.concatenate([ys_ref[0, :, :half], ys_ref[1, :, :half]], axis=1)
    return jnp.maximum(dis_ref[...] * (acc + y) + bias_ref[...], 0.0)


def _tc_b1_body(acc_ref, ys_ref, dis_ref, b0_ref, h_ref, stats_ref):
    i = pl.program_id(0)
    h = _finish_layer(acc_ref, ys_ref, dis_ref, b0_ref, 96)
    h_ref[...] = h
    s = jnp.sum(h, axis=0, keepdims=True)
    s2 = jnp.sum(h * h, axis=0, keepdims=True)
    contrib = jnp.concatenate([s, s2], axis=0)

    @pl.when(i == 0)
    def _():
        stats_ref[...] = contrib

    @pl.when(i > 0)
    def _():
        stats_ref[...] += contrib


def _tc_b2_body(h_ref, dis_ref, stats_ref, g_ref, b_ref, w_ref, ys_ref):
    wp, bias = _bn_fold(stats_ref, g_ref, b_ref, w_ref)
    y1 = (_dot(h_ref[...], wp) + bias) * dis_ref[...]
    ys_ref[0] = y1[:, :DH]
    ys_ref[1] = y1[:, DH:]


def _tc_c_body(acc_ref, ys_ref, dis_ref, b1_ref, batch_ref,
               l1w_ref, l1b_ref, l2w_ref, l2b_ref, pool_ref, out_ref):
    i = pl.program_id(0)
    h = _finish_layer(acc_ref, ys_ref, dis_ref, b1_ref, DH)
    gids = jax.lax.broadcasted_iota(jnp.int32, (BLK, 16), 1)
    onehot = (batch_ref[...] == gids).astype(jnp.float32)
    hext = jnp.concatenate([h, jnp.ones((BLK, 1), jnp.float32)], axis=1)
    contrib = jax.lax.dot_general(onehot, hext, (((0,), (0,)), ((), ())),
                                  precision=_HIGH,
                                  preferred_element_type=jnp.float32)

    @pl.when(i == 0)
    def _():
        pool_ref[...] = contrib

    @pl.when(i > 0)
    def _():
        pool_ref[...] += contrib

    @pl.when(i == NBLK - 1)
    def _():
        pooled = pool_ref[...]
        p = pooled[:, :256] / jnp.maximum(pooled[:, 256:257], 1.0)
        o = _dot(p, l1w_ref[...]) + l1b_ref[...]
        o = _dot(o, l2w_ref[...]) + l2b_ref[...]
        out_ref[...] = o


def kernel(x, edge_index, batch, bn0_g, bn0_b, W0, b0, bn1_g, bn1_b, W1, b1,
           l1W, l1b, l2W, l2b):
    src = edge_index[0]
    dst = edge_index[1]

    nch = E // NS // K
    src_r = src.reshape(NS, nch, K)
    srcg = jnp.concatenate([src_r, src_r + N], axis=0)
    dst_r = dst.reshape(NS, nch, K)
    dst_deg = dst.reshape(NC * NS, E // (NC * NS) // K, K)

    zeros_row = jnp.zeros((RPT, DH), jnp.float32)
    ones_row = jnp.ones((K, DH), jnp.float32)

    deg2 = _make_deg_kernel()(dst_deg, zeros_row, ones_row)

    d0, d1, d2 = 128, 192, 256
    row = lambda i: (i, 0)
    stk = lambda i: (0, i, 0)
    rep2 = pl.BlockSpec((2, d0), lambda i: (0, 0))
    col = pl.BlockSpec((BLK, 1), row)

    stats0 = _tc_stats(x, d0)
    ys0, dis = pl.pallas_call(
        _tc_a_body,
        grid=(NBLK,),
        in_specs=[pl.BlockSpec((BLK, d0), row),
                  pl.BlockSpec((2, BLK, DH), stk),
                  rep2,
                  pl.BlockSpec((1, d0), lambda i: (0, 0)),
                  pl.BlockSpec((1, d0), lambda i: (0, 0)),
                  pl.BlockSpec((d0, d1), lambda i: (0, 0))],
        out_specs=(pl.BlockSpec((2, BLK, DH), stk), col),
        out_shape=(jax.ShapeDtypeStruct((2, N, DH), jnp.float32),
                   jax.ShapeDtypeStruct((N, 1), jnp.float32)),
    )(x, deg2, stats0, bn0_g.reshape(1, d0), bn0_b.reshape(1, d0), W0)

    acc0 = _make_agg_kernel()(ys0.reshape(2 * N, DH), srcg, dst_r, zeros_row)

    h0, stats1 = pl.pallas_call(
        _tc_b1_body,
        grid=(NBLK,),
        in_specs=[pl.BlockSpec((2, BLK, DH), stk),
                  pl.BlockSpec((2, BLK, DH), stk),
                  col,
                  pl.BlockSpec((1, d1), lambda i: (0, 0))],
        out_specs=(pl.BlockSpec((BLK, d1), row),
                   pl.BlockSpec((2, d1), lambda i: (0, 0))),
        out_shape=(jax.ShapeDtypeStruct((N, d1), jnp.float32),
                   jax.ShapeDtypeStruct((2, d1), jnp.float32)),
    )(acc0, ys0, dis, b0.reshape(1, d1))

    ys1 = pl.pallas_call(
        _tc_b2_body,
        grid=(NBLK,),
        in_specs=[pl.BlockSpec((BLK, d1), row),
                  col,
                  pl.BlockSpec((2, d1), lambda i: (0, 0)),
                  pl.BlockSpec((1, d1), lambda i: (0, 0)),
                  pl.BlockSpec((1, d1), lambda i: (0, 0)),
                  pl.BlockSpec((d1, d2), lambda i: (0, 0))],
        out_specs=pl.BlockSpec((2, BLK, DH), stk),
        out_shape=jax.ShapeDtypeStruct((2, N, DH), jnp.float32),
    )(h0, dis, stats1, bn1_g.reshape(1, d1), bn1_b.reshape(1, d1), W1)

    acc1 = _make_agg_kernel()(ys1.reshape(2 * N, DH), srcg, dst_r, zeros_row)

    _, out = pl.pallas_call(
        _tc_c_body,
        grid=(NBLK,),
        in_specs=[pl.BlockSpec((2, BLK, DH), stk),
                  pl.BlockSpec((2, BLK, DH), stk),
                  col,
                  pl.BlockSpec((1, d2), lambda i: (0, 0)),
                  pl.BlockSpec((BLK, 1), row),
                  pl.BlockSpec((d2, d2 // 4), lambda i: (0, 0)),
                  pl.BlockSpec((1, d2 // 4), lambda i: (0, 0)),
                  pl.BlockSpec((d2 // 4, 10), lambda i: (0, 0)),
                  pl.BlockSpec((1, 10), lambda i: (0, 0))],
        out_specs=(pl.BlockSpec((16, d2 + 1), lambda i: (0, 0)),
                   pl.BlockSpec((16, 10), lambda i: (0, 0))),
        out_shape=(jax.ShapeDtypeStruct((16, d2 + 1), jnp.float32),
                   jax.ShapeDtypeStruct((16, 10), jnp.float32)),
    )(acc1, ys1, dis, b1.reshape(1, d2), batch.reshape(N, 1),
      l1W, l1b.reshape(1, d2 // 4), l2W, l2b.reshape(1, 10))

    return out


# speedup vs baseline: 19.1401x; 1.1353x over previous
import functools

import jax
import jax.numpy as jnp
from jax import lax
from jax.experimental import pallas as pl
from jax.experimental.pallas import tpu as pltpu
from jax.experimental.pallas import tpu_sc as plsc

N = 10000
NPAD = 10240
E = 320000
NS = 16
NC = 2
K = 125
RPT = NPAD // NS

_HIGH = jax.lax.Precision.HIGHEST
DH = 128
CB = 16


def _dot(a, b):
    return jax.lax.dot_general(a, b, (((1,), (0,)), ((), ())),
                               precision=_HIGH,
                               preferred_element_type=jnp.float32)


def _make_deg_kernel():
    nch = E // (NC * NS) // K
    mesh = plsc.VectorSubcoreMesh(core_axis_name="c", subcore_axis_name="s")

    @functools.partial(
        pl.kernel,
        out_type=jax.ShapeDtypeStruct((NC, NPAD, DH), jnp.float32),
        mesh=mesh,
        scratch_types=[
            pltpu.VMEM((nch, K), jnp.int32),
            pltpu.VMEM((K, DH), jnp.float32),
            pltpu.VMEM_SHARED((NPAD, DH), jnp.float32),
        ],
    )
    def deg_kernel(dst_hbm, zeros_hbm, ones_hbm, out_hbm, dst_v, ones_v, deg_sh):
        c = lax.axis_index("c")
        s = lax.axis_index("s")
        w = c * NS + s
        pltpu.sync_copy(dst_hbm.at[w], dst_v)
        pltpu.sync_copy(ones_hbm, ones_v)
        pltpu.sync_copy(zeros_hbm, deg_sh.at[pl.ds(s * RPT, RPT)])
        plsc.subcore_barrier()

        def body(j, carry):
            pltpu.sync_copy(ones_v, deg_sh.at[dst_v.at[j]], add=True)
            return carry

        lax.fori_loop(0, nch, body, 0)
        plsc.subcore_barrier()
        pltpu.sync_copy(deg_sh.at[pl.ds(s * RPT, RPT)],
                        out_hbm.at[c, pl.ds(s * RPT, RPT)])

    return deg_kernel


def _make_agg_kernel():
    nch = E // NS // K
    mesh = plsc.VectorSubcoreMesh(core_axis_name="c", subcore_axis_name="s")

    nb = nch // CB

    @functools.partial(
        pl.kernel,
        out_type=jax.ShapeDtypeStruct((NC, NPAD, DH), jnp.float32),
        mesh=mesh,
        scratch_types=[
            pltpu.VMEM((2, CB, K), jnp.int32),
            pltpu.VMEM((2, CB, K), jnp.int32),
            pltpu.VMEM((2, K, DH), jnp.float32),
            pltpu.VMEM_SHARED((NPAD, DH), jnp.float32),
            pltpu.SemaphoreType.DMA,
            pltpu.SemaphoreType.DMA,
            pltpu.SemaphoreType.DMA,
            pltpu.SemaphoreType.DMA,
            pltpu.SemaphoreType.DMA,
        ],
    )
    def agg_kernel(ys_hbm, srcg_hbm, dst_hbm, zeros_hbm, out_hbm,
                   src_v, dst_v, rows_v, acc_sh, gs0, gs1, ss0, ss1, isem):
        c = lax.axis_index("c")
        s = lax.axis_index("s")
        w = c * NS + s
        pltpu.sync_copy(srcg_hbm.at[w, pl.ds(0, CB)], src_v.at[0])
        pltpu.sync_copy(dst_hbm.at[s, pl.ds(0, CB)], dst_v.at[0])
        pltpu.sync_copy(zeros_hbm, acc_sh.at[pl.ds(s * RPT, RPT)])
        plsc.subcore_barrier()
        gsem = (gs0, gs1)
        ssem = (ss0, ss1)

        def outer(b, carry):
            bb = b % 2

            @pl.when(b + 1 < nb)
            def _():
                pltpu.async_copy(srcg_hbm.at[w, pl.ds((b + 1) * CB, CB)],
                                 src_v.at[1 - bb], isem)
                pltpu.async_copy(dst_hbm.at[s, pl.ds((b + 1) * CB, CB)],
                                 dst_v.at[1 - bb], isem)

            dg = [None, None]
            dsc = [None, None]
            dg[0] = pltpu.async_copy(ys_hbm.at[src_v.at[bb, 0]],
                                     rows_v.at[0], gsem[0])
            for jj in range(CB):
                rb = jj % 2
                if jj + 1 < CB:
                    if dsc[1 - rb] is not None:
                        dsc[1 - rb].wait()
                    dg[1 - rb] = pltpu.async_copy(
                        ys_hbm.at[src_v.at[bb, jj + 1]],
                        rows_v.at[1 - rb], gsem[1 - rb])
                dg[rb].wait()
                dsc[rb] = pltpu.async_copy(
                    rows_v.at[rb], acc_sh.at[dst_v.at[bb, jj]],
                    ssem[rb], add=True)
            dsc[0].wait()
            dsc[1].wait()

            @pl.when(b + 1 < nb)
            def _():
                pltpu.make_async_copy(srcg_hbm.at[w, pl.ds((b + 1) * CB, CB)],
                                      src_v.at[1 - bb], isem).wait()
                pltpu.make_async_copy(dst_hbm.at[s, pl.ds((b + 1) * CB, CB)],
                                      dst_v.at[1 - bb], isem).wait()

            return carry

        lax.fori_loop(0, nb, outer, 0)
        plsc.subcore_barrier()
        pltpu.sync_copy(acc_sh.at[pl.ds(s * RPT, RPT)],
                        out_hbm.at[c, pl.ds(s * RPT, RPT)])

    return agg_kernel


NBLK = 5
BLK = N // NBLK


def _stats_body(x_ref, out_ref):
    i = pl.program_id(0)
    xv = x_ref[...]
    s = jnp.sum(xv, axis=0, keepdims=True)
    s2 = jnp.sum(xv * xv, axis=0, keepdims=True)
    contrib = jnp.concatenate([s, s2], axis=0)

    @pl.when(i == 0)
    def _():
        out_ref[...] = contrib

    @pl.when(i > 0)
    def _():
        out_ref[...] += contrib


def _tc_stats(x, d):
    return pl.pallas_call(
        _stats_body,
        grid=(NBLK,),
        in_specs=[pl.BlockSpec((BLK, d), lambda i: (i, 0))],
        out_specs=pl.BlockSpec((2, d), lambda i: (0, 0)),
        out_shape=jax.ShapeDtypeStruct((2, d), jnp.float32),
    )(x)


def _bn_fold(stats_ref, g_ref, b_ref, w_ref):
    mean = stats_ref[0:1, :] / N
    var = stats_ref[1:2, :] / N - mean * mean
    scale = jax.lax.rsqrt(var + 1e-5) * g_ref[...]
    wp = w_ref[...] * scale.T
    bias = _dot(b_ref[...] - mean * scale, w_ref[...])
    return wp, bias


def _tc_a_body(x_ref, deg_ref, stats_ref, g_ref, b_ref, w_ref,
               ys_ref, dis_ref):
    wp, bias = _bn_fold(stats_ref, g_ref, b_ref, w_ref)
    deg = deg_ref[0, :, 0:1] + deg_ref[1, :, 0:1] + 1.0
    dis = jax.lax.rsqrt(deg)
    y = (_dot(x_ref[...], wp) + bias) * dis
    pad = jnp.zeros((BLK, DH - 96), jnp.float32)
    ys_ref[0] = jnp.concatenate([y[:, :96], pad], axis=1)
    ys_ref[1] = jnp.concatenate([y[:, 96:], pad], axis=1)
    dis_ref[...] = dis


def _finish_layer(acc_ref, ys_ref, dis_ref, bias_ref, half):
    acc = jnp.concatenate([acc_ref[0, :, :half], acc_ref[1, :, :half]], axis=1)
    y = jnp.concatenate([ys_ref[0, :, :half], ys_ref[1, :, :half]], axis=1)
    return jnp.maximum(dis_ref[...] * (acc + y) + bias_ref[...], 0.0)


def _tc_b1_body(acc_ref, ys_ref, dis_ref, b0_ref, h_ref, stats_ref):
    i = pl.program_id(0)
    h = _finish_layer(acc_ref, ys_ref, dis_ref, b0_ref, 96)
    h_ref[...] = h
    s = jnp.sum(h, axis=0, keepdims=True)
    s2 = jnp.sum(h * h, axis=0, keepdims=True)
    contrib = jnp.concatenate([s, s2], axis=0)

    @pl.when(i == 0)
    def _():
        stats_ref[...] = contrib

    @pl.when(i > 0)
    def _():
        stats_ref[...] += contrib


def _tc_b2_body(h_ref, dis_ref, stats_ref, g_ref, b_ref, w_ref, ys_ref):
    wp, bias = _bn_fold(stats_ref, g_ref, b_ref, w_ref)
    y1 = (_dot(h_ref[...], wp) + bias) * dis_ref[...]
    ys_ref[0] = y1[:, :DH]
    ys_ref[1] = y1[:, DH:]


def _tc_c_body(acc_ref, ys_ref, dis_ref, b1_ref, batch_ref,
               l1w_ref, l1b_ref, l2w_ref, l2b_ref, pool_ref, out_ref):
    i = pl.program_id(0)
    h = _finish_layer(acc_ref, ys_ref, dis_ref, b1_ref, DH)
    gids = jax.lax.broadcasted_iota(jnp.int32, (BLK, 16), 1)
    onehot = (batch_ref[...] == gids).astype(jnp.float32)
    hext = jnp.concatenate([h, jnp.ones((BLK, 1), jnp.float32)], axis=1)
    contrib = jax.lax.dot_general(onehot, hext, (((0,), (0,)), ((), ())),
                                  precision=_HIGH,
                                  preferred_element_type=jnp.float32)

    @pl.when(i == 0)
    def _():
        pool_ref[...] = contrib

    @pl.when(i > 0)
    def _():
        pool_ref[...] += contrib

    @pl.when(i == NBLK - 1)
    def _():
        pooled = pool_ref[...]
        p = pooled[:, :256] / jnp.maximum(pooled[:, 256:257], 1.0)
        o = _dot(p, l1w_ref[...]) + l1b_ref[...]
        o = _dot(o, l2w_ref[...]) + l2b_ref[...]
        out_ref[...] = o


def kernel(x, edge_index, batch, bn0_g, bn0_b, W0, b0, bn1_g, bn1_b, W1, b1,
           l1W, l1b, l2W, l2b):
    src = edge_index[0]
    dst = edge_index[1]

    nch = E // NS // K
    src_r = src.reshape(NS, nch, K)
    srcg = jnp.concatenate([src_r, src_r + N], axis=0)
    dst_r = dst.reshape(NS, nch, K)
    dst_deg = dst.reshape(NC * NS, E // (NC * NS) // K, K)

    zeros_row = jnp.zeros((RPT, DH), jnp.float32)
    ones_row = jnp.ones((K, DH), jnp.float32)

    deg2 = _make_deg_kernel()(dst_deg, zeros_row, ones_row)

    d0, d1, d2 = 128, 192, 256
    row = lambda i: (i, 0)
    stk = lambda i: (0, i, 0)
    rep2 = pl.BlockSpec((2, d0), lambda i: (0, 0))
    col = pl.BlockSpec((BLK, 1), row)

    stats0 = _tc_stats(x, d0)
    ys0, dis = pl.pallas_call(
        _tc_a_body,
        grid=(NBLK,),
        in_specs=[pl.BlockSpec((BLK, d0), row),
                  pl.BlockSpec((2, BLK, DH), stk),
                  rep2,
                  pl.BlockSpec((1, d0), lambda i: (0, 0)),
                  pl.BlockSpec((1, d0), lambda i: (0, 0)),
                  pl.BlockSpec((d0, d1), lambda i: (0, 0))],
        out_specs=(pl.BlockSpec((2, BLK, DH), stk), col),
        out_shape=(jax.ShapeDtypeStruct((2, N, DH), jnp.float32),
                   jax.ShapeDtypeStruct((N, 1), jnp.float32)),
    )(x, deg2, stats0, bn0_g.reshape(1, d0), bn0_b.reshape(1, d0), W0)

    acc0 = _make_agg_kernel()(ys0.reshape(2 * N, DH), srcg, dst_r, zeros_row)

    h0, stats1 = pl.pallas_call(
        _tc_b1_body,
        grid=(NBLK,),
        in_specs=[pl.BlockSpec((2, BLK, DH), stk),
                  pl.BlockSpec((2, BLK, DH), stk),
                  col,
                  pl.BlockSpec((1, d1), lambda i: (0, 0))],
        out_specs=(pl.BlockSpec((BLK, d1), row),
                   pl.BlockSpec((2, d1), lambda i: (0, 0))),
        out_shape=(jax.ShapeDtypeStruct((N, d1), jnp.float32),
                   jax.ShapeDtypeStruct((2, d1), jnp.float32)),
    )(acc0, ys0, dis, b0.reshape(1, d1))

    ys1 = pl.pallas_call(
        _tc_b2_body,
        grid=(NBLK,),
        in_specs=[pl.BlockSpec((BLK, d1), row),
                  col,
                  pl.BlockSpec((2, d1), lambda i: (0, 0)),
                  pl.BlockSpec((1, d1), lambda i: (0, 0)),
                  pl.BlockSpec((1, d1), lambda i: (0, 0)),
                  pl.BlockSpec((d1, d2), lambda i: (0, 0))],
        out_specs=pl.BlockSpec((2, BLK, DH), stk),
        out_shape=jax.ShapeDtypeStruct((2, N, DH), jnp.float32),
    )(h0, dis, stats1, bn1_g.reshape(1, d1), bn1_b.reshape(1, d1), W1)

    acc1 = _make_agg_kernel()(ys1.reshape(2 * N, DH), srcg, dst_r, zeros_row)

    _, out = pl.pallas_call(
        _tc_c_body,
        grid=(NBLK,),
        in_specs=[pl.BlockSpec((2, BLK, DH), stk),
                  pl.BlockSpec((2, BLK, DH), stk),
                  col,
                  pl.BlockSpec((1, d2), lambda i: (0, 0)),
                  pl.BlockSpec((BLK, 1), row),
                  pl.BlockSpec((d2, d2 // 4), lambda i: (0, 0)),
                  pl.BlockSpec((1, d2 // 4), lambda i: (0, 0)),
                  pl.BlockSpec((d2 // 4, 10), lambda i: (0, 0)),
                  pl.BlockSpec((1, 10), lambda i: (0, 0))],
        out_specs=(pl.BlockSpec((16, d2 + 1), lambda i: (0, 0)),
                   pl.BlockSpec((16, 10), lambda i: (0, 0))),
        out_shape=(jax.ShapeDtypeStruct((16, d2 + 1), jnp.float32),
                   jax.ShapeDtypeStruct((16, 10), jnp.float32)),
    )(acc1, ys1, dis, b1.reshape(1, d2), batch.reshape(N, 1),
      l1W, l1b.reshape(1, d2 // 4), l2W, l2b.reshape(1, 10))

    return out


# speedup vs baseline: 19.6877x; 1.0286x over previous
import functools

import jax
import jax.numpy as jnp
from jax import lax
from jax.experimental import pallas as pl
from jax.experimental.pallas import tpu as pltpu
from jax.experimental.pallas import tpu_sc as plsc

N = 10000
NPAD = 10240
E = 320000
NS = 16
NC = 2
K = 125
KA = 100
RPT = NPAD // NS

_HIGH = jax.lax.Precision.HIGHEST
DH = 128
CB = 20


def _dot(a, b):
    return jax.lax.dot_general(a, b, (((1,), (0,)), ((), ())),
                               precision=_HIGH,
                               preferred_element_type=jnp.float32)


def _make_deg_kernel():
    nch = E // (NC * NS) // K
    mesh = plsc.VectorSubcoreMesh(core_axis_name="c", subcore_axis_name="s")

    @functools.partial(
        pl.kernel,
        out_type=jax.ShapeDtypeStruct((NC, NPAD, DH), jnp.float32),
        mesh=mesh,
        scratch_types=[
            pltpu.VMEM((nch, K), jnp.int32),
            pltpu.VMEM((K, DH), jnp.float32),
            pltpu.VMEM_SHARED((NPAD, DH), jnp.float32),
            pltpu.SemaphoreType.DMA,
        ],
    )
    def deg_kernel(dst_hbm, zeros_hbm, ones_hbm, out_hbm, dst_v, ones_v, deg_sh,
                   ssem):
        c = lax.axis_index("c")
        s = lax.axis_index("s")
        w = c * NS + s
        pltpu.sync_copy(dst_hbm.at[w], dst_v)
        pltpu.sync_copy(ones_hbm, ones_v)
        pltpu.sync_copy(zeros_hbm, deg_sh.at[pl.ds(s * RPT, RPT)])
        plsc.subcore_barrier()

        kf = 16

        def body(b, carry):
            descs = [pltpu.async_copy(ones_v, deg_sh.at[dst_v.at[b * kf + jj]],
                                      ssem, add=True)
                     for jj in range(kf)]
            for dd in descs:
                dd.wait()
            return carry

        lax.fori_loop(0, nch // kf, body, 0)
        plsc.subcore_barrier()
        pltpu.sync_copy(deg_sh.at[pl.ds(s * RPT, RPT)],
                        out_hbm.at[c, pl.ds(s * RPT, RPT)])

    return deg_kernel


def _make_agg_kernel():
    nch = E // NS // KA
    mesh = plsc.VectorSubcoreMesh(core_axis_name="c", subcore_axis_name="s")

    nb = nch // CB

    @functools.partial(
        pl.kernel,
        out_type=jax.ShapeDtypeStruct((NC, NPAD, DH), jnp.float32),
        mesh=mesh,
        scratch_types=[
            pltpu.VMEM((CB, KA), jnp.int32),
            pltpu.VMEM((CB, KA), jnp.int32),
            pltpu.VMEM((3, KA, DH), jnp.float32),
            pltpu.VMEM_SHARED((NPAD, DH), jnp.float32),
            pltpu.SemaphoreType.DMA,
            pltpu.SemaphoreType.DMA,
            pltpu.SemaphoreType.DMA,
            pltpu.SemaphoreType.DMA,
            pltpu.SemaphoreType.DMA,
            pltpu.SemaphoreType.DMA,
        ],
    )
    def agg_kernel(ys_hbm, srcg_hbm, dst_hbm, zeros_hbm, out_hbm,
                   src_v, dst_v, rows_v, acc_sh, gs0, gs1, gs2,
                   ss0, ss1, ss2):
        c = lax.axis_index("c")
        s = lax.axis_index("s")
        w = c * NS + s
        pltpu.sync_copy(zeros_hbm, acc_sh.at[pl.ds(s * RPT, RPT)])
        plsc.subcore_barrier()
        gsem = (gs0, gs1, gs2)
        ssem = (ss0, ss1, ss2)

        def outer(b, carry):
            pltpu.sync_copy(srcg_hbm.at[w, b], src_v)
            pltpu.sync_copy(dst_hbm.at[s, b], dst_v)

            dg = [None, None, None]
            dsc = [None, None, None]
            dg[0] = pltpu.async_copy(ys_hbm.at[src_v.at[0]],
                                     rows_v.at[0], gsem[0])
            dg[1] = pltpu.async_copy(ys_hbm.at[src_v.at[1]],
                                     rows_v.at[1], gsem[1])
            for jj in range(CB):
                rb = jj % 3
                if jj + 2 < CB:
                    nxt = (jj + 2) % 3
                    if dsc[nxt] is not None:
                        dsc[nxt].wait()
                    dg[nxt] = pltpu.async_copy(
                        ys_hbm.at[src_v.at[jj + 2]],
                        rows_v.at[nxt], gsem[nxt])
                dg[rb].wait()
                dsc[rb] = pltpu.async_copy(
                    rows_v.at[rb], acc_sh.at[dst_v.at[jj]],
                    ssem[rb], add=True)
            for dd in dsc:
                if dd is not None:
                    dd.wait()

            return carry

        lax.fori_loop(0, nb, outer, 0)
        plsc.subcore_barrier()
        pltpu.sync_copy(acc_sh.at[pl.ds(s * RPT, RPT)],
                        out_hbm.at[c, pl.ds(s * RPT, RPT)])

    return agg_kernel


NBLK = 5
BLK = N // NBLK


def _stats_body(x_ref, out_ref):
    i = pl.program_id(0)
    xv = x_ref[...]
    s = jnp.sum(xv, axis=0, keepdims=True)
    s2 = jnp.sum(xv * xv, axis=0, keepdims=True)
    contrib = jnp.concatenate([s, s2], axis=0)

    @pl.when(i == 0)
    def _():
        out_ref[...] = contrib

    @pl.when(i > 0)
    def _():
        out_ref[...] += contrib


def _tc_stats(x, d):
    return pl.pallas_call(
        _stats_body,
        grid=(NBLK,),
        in_specs=[pl.BlockSpec((BLK, d), lambda i: (i, 0))],
        out_specs=pl.BlockSpec((2, d), lambda i: (0, 0)),
        out_shape=jax.ShapeDtypeStruct((2, d), jnp.float32),
    )(x)


def _bn_fold(stats_ref, g_ref, b_ref, w_ref):
    mean = stats_ref[0:1, :] / N
    var = stats_ref[1:2, :] / N - mean * mean
    scale = jax.lax.rsqrt(var + 1e-5) * g_ref[...]
    wp = w_ref[...] * scale.T
    bias = _dot(b_ref[...] - mean * scale, w_ref[...])
    return wp, bias


def _tc_a_body(x_ref, deg_ref, stats_ref, g_ref, b_ref, w_ref,
               ys_ref, dis_ref):
    wp, bias = _bn_fold(stats_ref, g_ref, b_ref, w_ref)
    deg = deg_ref[0, :, 0:1] + deg_ref[1, :, 0:1] + 1.0
    dis = jax.lax.rsqrt(deg)
    y = (_dot(x_ref[...], wp) + bias) * dis
    pad = jnp.zeros((BLK, DH - 96), jnp.float32)
    ys_ref[0] = jnp.concatenate([y[:, :96], pad], axis=1)
    ys_ref[1] = jnp.concatenate([y[:, 96:], pad], axis=1)
    dis_ref[...] = dis


def _finish_layer(acc_ref, ys_ref, dis_ref, bias_ref, half):
    acc = jnp.concatenate([acc_ref[0, :, :half], acc_ref[1, :, :half]], axis=1)
    y = jnp.concatenate([ys_ref[0, :, :half], ys_ref[1, :, :half]], axis=1)
    return jnp.maximum(dis_ref[...] * (acc + y) + bias_ref[...], 0.0)


def _tc_b1_body(acc_ref, ys_ref, dis_ref, b0_ref, h_ref, stats_ref):
    i = pl.program_id(0)
    h = _finish_layer(acc_ref, ys_ref, dis_ref, b0_ref, 96)
    h_ref[...] = h
    s = jnp.sum(h, axis=0, keepdims=True)
    s2 = jnp.sum(h * h, axis=0, keepdims=True)
    contrib = jnp.concatenate([s, s2], axis=0)

    @pl.when(i == 0)
    def _():
        stats_ref[...] = contrib

    @pl.when(i > 0)
    def _():
        stats_ref[...] += contrib


def _tc_b2_body(h_ref, dis_ref, stats_ref, g_ref, b_ref, w_ref, ys_ref):
    wp, bias = _bn_fold(stats_ref, g_ref, b_ref, w_ref)
    y1 = (_dot(h_ref[...], wp) + bias) * dis_ref[...]
    ys_ref[0] = y1[:, :DH]
    ys_ref[1] = y1[:, DH:]


def _tc_c_body(acc_ref, ys_ref, dis_ref, b1_ref, batch_ref,
               l1w_ref, l1b_ref, l2w_ref, l2b_ref, pool_ref, out_ref):
    i = pl.program_id(0)
    h = _finish_layer(acc_ref, ys_ref, dis_ref, b1_ref, DH)
    gids = jax.lax.broadcasted_iota(jnp.int32, (BLK, 16), 1)
    onehot = (batch_ref[...] == gids).astype(jnp.float32)
    hext = jnp.concatenate([h, jnp.ones((BLK, 1), jnp.float32)], axis=1)
    contrib = jax.lax.dot_general(onehot, hext, (((0,), (0,)), ((), ())),
                                  precision=_HIGH,
                                  preferred_element_type=jnp.float32)

    @pl.when(i == 0)
    def _():
        pool_ref[...] = contrib

    @pl.when(i > 0)
    def _():
        pool_ref[...] += contrib

    @pl.when(i == NBLK - 1)
    def _():
        pooled = pool_ref[...]
        p = pooled[:, :256] / jnp.maximum(pooled[:, 256:257], 1.0)
        o = _dot(p, l1w_ref[...]) + l1b_ref[...]
        o = _dot(o, l2w_ref[...]) + l2b_ref[...]
        out_ref[...] = o


def kernel(x, edge_index, batch, bn0_g, bn0_b, W0, b0, bn1_g, bn1_b, W1, b1,
           l1W, l1b, l2W, l2b):
    src = edge_index[0]
    dst = edge_index[1]

    nch = E // NS // KA
    nb = nch // CB
    src_r = src.reshape(NS, nb, CB, KA)
    srcg = jnp.concatenate([src_r, src_r + N], axis=0)
    dst_r = dst.reshape(NS, nb, CB, KA)
    dst_deg = dst.reshape(NC * NS, E // (NC * NS) // K, K)

    zeros_row = jnp.zeros((RPT, DH), jnp.float32)
    ones_row = jnp.ones((K, DH), jnp.float32)

    deg2 = _make_deg_kernel()(dst_deg, zeros_row, ones_row)

    d0, d1, d2 = 128, 192, 256
    row = lambda i: (i, 0)
    stk = lambda i: (0, i, 0)
    rep2 = pl.BlockSpec((2, d0), lambda i: (0, 0))
    col = pl.BlockSpec((BLK, 1), row)

    stats0 = _tc_stats(x, d0)
    ys0, dis = pl.pallas_call(
        _tc_a_body,
        grid=(NBLK,),
        in_specs=[pl.BlockSpec((BLK, d0), row),
                  pl.BlockSpec((2, BLK, DH), stk),
                  rep2,
                  pl.BlockSpec((1, d0), lambda i: (0, 0)),
                  pl.BlockSpec((1, d0), lambda i: (0, 0)),
                  pl.BlockSpec((d0, d1), lambda i: (0, 0))],
        out_specs=(pl.BlockSpec((2, BLK, DH), stk), col),
        out_shape=(jax.ShapeDtypeStruct((2, N, DH), jnp.float32),
                   jax.ShapeDtypeStruct((N, 1), jnp.float32)),
    )(x, deg2, stats0, bn0_g.reshape(1, d0), bn0_b.reshape(1, d0), W0)

    acc0 = _make_agg_kernel()(ys0.reshape(2 * N, DH), srcg, dst_r, zeros_row)

    h0, stats1 = pl.pallas_call(
        _tc_b1_body,
        grid=(NBLK,),
        in_specs=[pl.BlockSpec((2, BLK, DH), stk),
                  pl.BlockSpec((2, BLK, DH), stk),
                  col,
                  pl.BlockSpec((1, d1), lambda i: (0, 0))],
        out_specs=(pl.BlockSpec((BLK, d1), row),
                   pl.BlockSpec((2, d1), lambda i: (0, 0))),
        out_shape=(jax.ShapeDtypeStruct((N, d1), jnp.float32),
                   jax.ShapeDtypeStruct((2, d1), jnp.float32)),
    )(acc0, ys0, dis, b0.reshape(1, d1))

    ys1 = pl.pallas_call(
        _tc_b2_body,
        grid=(NBLK,),
        in_specs=[pl.BlockSpec((BLK, d1), row),
                  col,
                  pl.BlockSpec((2, d1), lambda i: (0, 0)),
                  pl.BlockSpec((1, d1), lambda i: (0, 0)),
                  pl.BlockSpec((1, d1), lambda i: (0, 0)),
                  pl.BlockSpec((d1, d2), lambda i: (0, 0))],
        out_specs=pl.BlockSpec((2, BLK, DH), stk),
        out_shape=jax.ShapeDtypeStruct((2, N, DH), jnp.float32),
    )(h0, dis, stats1, bn1_g.reshape(1, d1), bn1_b.reshape(1, d1), W1)

    acc1 = _make_agg_kernel()(ys1.reshape(2 * N, DH), srcg, dst_r, zeros_row)

    _, out = pl.pallas_call(
        _tc_c_body,
        grid=(NBLK,),
        in_specs=[pl.BlockSpec((2, BLK, DH), stk),
                  pl.BlockSpec((2, BLK, DH), stk),
                  col,
                  pl.BlockSpec((1, d2), lambda i: (0, 0)),
                  pl.BlockSpec((BLK, 1), row),
                  pl.BlockSpec((d2, d2 // 4), lambda i: (0, 0)),
                  pl.BlockSpec((1, d2 // 4), lambda i: (0, 0)),
                  pl.BlockSpec((d2 // 4, 10), lambda i: (0, 0)),
                  pl.BlockSpec((1, 10), lambda i: (0, 0))],
        out_specs=(pl.BlockSpec((16, d2 + 1), lambda i: (0, 0)),
                   pl.BlockSpec((16, 10), lambda i: (0, 0))),
        out_shape=(jax.ShapeDtypeStruct((16, d2 + 1), jnp.float32),
                   jax.ShapeDtypeStruct((16, 10), jnp.float32)),
    )(acc1, ys1, dis, b1.reshape(1, d2), batch.reshape(N, 1),
      l1W, l1b.reshape(1, d2 // 4), l2W, l2b.reshape(1, 10))

    return out


# speedup vs baseline: 19.7212x; 1.0017x over previous
import functools

import jax
import jax.numpy as jnp
from jax import lax
from jax.experimental import pallas as pl
from jax.experimental.pallas import tpu as pltpu
from jax.experimental.pallas import tpu_sc as plsc

N = 10000
NPAD = 10240
E = 320000
NS = 16
NC = 2
K = 125
KA = 100
RPT = NPAD // NS

_HIGH = jax.lax.Precision.HIGHEST
DH = 128
CB = 20


def _dot(a, b):
    return jax.lax.dot_general(a, b, (((1,), (0,)), ((), ())),
                               precision=_HIGH,
                               preferred_element_type=jnp.float32)


def _make_deg_kernel():
    nch = E // (NC * NS) // K
    mesh = plsc.VectorSubcoreMesh(core_axis_name="c", subcore_axis_name="s")

    @functools.partial(
        pl.kernel,
        out_type=jax.ShapeDtypeStruct((NC, NPAD, DH), jnp.float32),
        mesh=mesh,
        scratch_types=[
            pltpu.VMEM((nch, K), jnp.int32),
            pltpu.VMEM((K, DH), jnp.float32),
            pltpu.VMEM_SHARED((NPAD, DH), jnp.float32),
            pltpu.SemaphoreType.DMA,
        ],
    )
    def deg_kernel(dst_hbm, zeros_hbm, ones_hbm, out_hbm, dst_v, ones_v, deg_sh,
                   ssem):
        c = lax.axis_index("c")
        s = lax.axis_index("s")
        w = c * NS + s
        pltpu.sync_copy(dst_hbm.at[w], dst_v)
        pltpu.sync_copy(ones_hbm, ones_v)
        pltpu.sync_copy(zeros_hbm, deg_sh.at[pl.ds(s * RPT, RPT)])
        plsc.subcore_barrier()

        kf = 16

        def body(b, carry):
            descs = [pltpu.async_copy(ones_v, deg_sh.at[dst_v.at[b * kf + jj]],
                                      ssem, add=True)
                     for jj in range(kf)]
            for dd in descs:
                dd.wait()
            return carry

        lax.fori_loop(0, nch // kf, body, 0)
        plsc.subcore_barrier()
        pltpu.sync_copy(deg_sh.at[pl.ds(s * RPT, RPT)],
                        out_hbm.at[c, pl.ds(s * RPT, RPT)])

    return deg_kernel


def _make_agg_kernel():
    nch = E // NS // KA
    mesh = plsc.VectorSubcoreMesh(core_axis_name="c", subcore_axis_name="s")

    nb = nch // CB

    @functools.partial(
        pl.kernel,
        out_type=jax.ShapeDtypeStruct((NC, NPAD, DH), jnp.float32),
        mesh=mesh,
        scratch_types=[
            pltpu.VMEM((CB, KA), jnp.int32),
            pltpu.VMEM((CB, KA), jnp.int32),
            pltpu.VMEM((3, KA, DH), jnp.float32),
            pltpu.VMEM_SHARED((NPAD, DH), jnp.float32),
            pltpu.SemaphoreType.DMA,
            pltpu.SemaphoreType.DMA,
            pltpu.SemaphoreType.DMA,
            pltpu.SemaphoreType.DMA,
            pltpu.SemaphoreType.DMA,
            pltpu.SemaphoreType.DMA,
        ],
    )
    def agg_kernel(ys_hbm, src_hbm, dst_hbm, zeros_hbm, out_hbm,
                   src_v, dst_v, rows_v, acc_sh, gs0, gs1, gs2,
                   ss0, ss1, ss2):
        c = lax.axis_index("c")
        s = lax.axis_index("s")
        ys_half = ys_hbm.at[c]
        pltpu.sync_copy(zeros_hbm, acc_sh.at[pl.ds(s * RPT, RPT)])
        plsc.subcore_barrier()
        gsem = (gs0, gs1, gs2)
        ssem = (ss0, ss1, ss2)

        def outer(b, carry):
            pltpu.sync_copy(src_hbm.at[s, b], src_v)
            pltpu.sync_copy(dst_hbm.at[s, b], dst_v)

            dg = [None, None, None]
            dsc = [None, None, None]
            dg[0] = pltpu.async_copy(ys_half.at[src_v.at[0]],
                                     rows_v.at[0], gsem[0])
            dg[1] = pltpu.async_copy(ys_half.at[src_v.at[1]],
                                     rows_v.at[1], gsem[1])
            for jj in range(CB):
                rb = jj % 3
                if jj + 2 < CB:
                    nxt = (jj + 2) % 3
                    if dsc[nxt] is not None:
                        dsc[nxt].wait()
                    dg[nxt] = pltpu.async_copy(
                        ys_half.at[src_v.at[jj + 2]],
                        rows_v.at[nxt], gsem[nxt])
                dg[rb].wait()
                dsc[rb] = pltpu.async_copy(
                    rows_v.at[rb], acc_sh.at[dst_v.at[jj]],
                    ssem[rb], add=True)
            for dd in dsc:
                if dd is not None:
                    dd.wait()

            return carry

        lax.fori_loop(0, nb, outer, 0)
        plsc.subcore_barrier()
        pltpu.sync_copy(acc_sh.at[pl.ds(s * RPT, RPT)],
                        out_hbm.at[c, pl.ds(s * RPT, RPT)])

    return agg_kernel


NBLK = 5
BLK = N // NBLK


def _stats_body(x_ref, out_ref):
    i = pl.program_id(0)
    xv = x_ref[...]
    s = jnp.sum(xv, axis=0, keepdims=True)
    s2 = jnp.sum(xv * xv, axis=0, keepdims=True)
    contrib = jnp.concatenate([s, s2], axis=0)

    @pl.when(i == 0)
    def _():
        out_ref[...] = contrib

    @pl.when(i > 0)
    def _():
        out_ref[...] += contrib


def _tc_stats(x, d):
    return pl.pallas_call(
        _stats_body,
        grid=(NBLK,),
        in_specs=[pl.BlockSpec((BLK, d), lambda i: (i, 0))],
        out_specs=pl.BlockSpec((2, d), lambda i: (0, 0)),
        out_shape=jax.ShapeDtypeStruct((2, d), jnp.float32),
    )(x)


def _bn_fold(stats_ref, g_ref, b_ref, w_ref):
    mean = stats_ref[0:1, :] / N
    var = stats_ref[1:2, :] / N - mean * mean
    scale = jax.lax.rsqrt(var + 1e-5) * g_ref[...]
    wp = w_ref[...] * scale.T
    bias = _dot(b_ref[...] - mean * scale, w_ref[...])
    return wp, bias


def _tc_xw_body(x_ref, stats_ref, g_ref, b_ref, w_ref, xw_ref):
    wp, bias = _bn_fold(stats_ref, g_ref, b_ref, w_ref)
    xw_ref[...] = _dot(x_ref[...], wp) + bias


def _tc_scale_body(xw_ref, deg_ref, ys_ref, dis_ref):
    deg = deg_ref[0, :, 0:1] + deg_ref[1, :, 0:1] + 1.0
    dis = jax.lax.rsqrt(deg)
    y = xw_ref[...] * dis
    pad = jnp.zeros((BLK, DH - 96), jnp.float32)
    ys_ref[0] = jnp.concatenate([y[:, :96], pad], axis=1)
    ys_ref[1] = jnp.concatenate([y[:, 96:], pad], axis=1)
    dis_ref[...] = dis


def _finish_layer(acc_ref, ys_ref, dis_ref, bias_ref, half):
    acc = jnp.concatenate([acc_ref[0, :, :half], acc_ref[1, :, :half]], axis=1)
    y = jnp.concatenate([ys_ref[0, :, :half], ys_ref[1, :, :half]], axis=1)
    return jnp.maximum(dis_ref[...] * (acc + y) + bias_ref[...], 0.0)


def _tc_b1_body(acc_ref, ys_ref, dis_ref, b0_ref, h_ref, stats_ref):
    i = pl.program_id(0)
    h = _finish_layer(acc_ref, ys_ref, dis_ref, b0_ref, 96)
    h_ref[...] = h
    s = jnp.sum(h, axis=0, keepdims=True)
    s2 = jnp.sum(h * h, axis=0, keepdims=True)
    contrib = jnp.concatenate([s, s2], axis=0)

    @pl.when(i == 0)
    def _():
        stats_ref[...] = contrib

    @pl.when(i > 0)
    def _():
        stats_ref[...] += contrib


def _tc_b2_body(h_ref, dis_ref, stats_ref, g_ref, b_ref, w_ref, ys_ref):
    wp, bias = _bn_fold(stats_ref, g_ref, b_ref, w_ref)
    y1 = (_dot(h_ref[...], wp) + bias) * dis_ref[...]
    ys_ref[0] = y1[:, :DH]
    ys_ref[1] = y1[:, DH:]


def _tc_c_body(acc_ref, ys_ref, dis_ref, b1_ref, batch_ref,
               l1w_ref, l1b_ref, l2w_ref, l2b_ref, pool_ref, out_ref):
    i = pl.program_id(0)
    h = _finish_layer(acc_ref, ys_ref, dis_ref, b1_ref, DH)
    gids = jax.lax.broadcasted_iota(jnp.int32, (BLK, 16), 1)
    onehot = (batch_ref[...] == gids).astype(jnp.float32)
    hext = jnp.concatenate([h, jnp.ones((BLK, 1), jnp.float32)], axis=1)
    contrib = jax.lax.dot_general(onehot, hext, (((0,), (0,)), ((), ())),
                                  precision=_HIGH,
                                  preferred_element_type=jnp.float32)

    @pl.when(i == 0)
    def _():
        pool_ref[...] = contrib

    @pl.when(i > 0)
    def _():
        pool_ref[...] += contrib

    @pl.when(i == NBLK - 1)
    def _():
        pooled = pool_ref[...]
        p = pooled[:, :256] / jnp.maximum(pooled[:, 256:257], 1.0)
        o = _dot(p, l1w_ref[...]) + l1b_ref[...]
        o = _dot(o, l2w_ref[...]) + l2b_ref[...]
        out_ref[...] = o


def kernel(x, edge_index, batch, bn0_g, bn0_b, W0, b0, bn1_g, bn1_b, W1, b1,
           l1W, l1b, l2W, l2b):
    src = edge_index[0]
    dst = edge_index[1]

    nch = E // NS // KA
    nb = nch // CB
    src_r = src.reshape(NS, nb, CB, KA)
    dst_r = dst.reshape(NS, nb, CB, KA)
    dst_deg = dst.reshape(NC * NS, E // (NC * NS) // K, K)

    zeros_row = jnp.zeros((RPT, DH), jnp.float32)
    ones_row = jnp.ones((K, DH), jnp.float32)

    deg2 = _make_deg_kernel()(dst_deg, zeros_row, ones_row)

    d0, d1, d2 = 128, 192, 256
    row = lambda i: (i, 0)
    stk = lambda i: (0, i, 0)
    rep2 = pl.BlockSpec((2, d0), lambda i: (0, 0))
    col = pl.BlockSpec((BLK, 1), row)

    stats0 = _tc_stats(x, d0)
    xw0 = pl.pallas_call(
        _tc_xw_body,
        grid=(NBLK,),
        in_specs=[pl.BlockSpec((BLK, d0), row),
                  rep2,
                  pl.BlockSpec((1, d0), lambda i: (0, 0)),
                  pl.BlockSpec((1, d0), lambda i: (0, 0)),
                  pl.BlockSpec((d0, d1), lambda i: (0, 0))],
        out_specs=pl.BlockSpec((BLK, d1), row),
        out_shape=jax.ShapeDtypeStruct((N, d1), jnp.float32),
    )(x, stats0, bn0_g.reshape(1, d0), bn0_b.reshape(1, d0), W0)

    ys0, dis = pl.pallas_call(
        _tc_scale_body,
        grid=(NBLK,),
        in_specs=[pl.BlockSpec((BLK, d1), row),
                  pl.BlockSpec((2, BLK, DH), stk)],
        out_specs=(pl.BlockSpec((2, BLK, DH), stk), col),
        out_shape=(jax.ShapeDtypeStruct((2, N, DH), jnp.float32),
                   jax.ShapeDtypeStruct((N, 1), jnp.float32)),
    )(xw0, deg2)

    acc0 = _make_agg_kernel()(ys0, src_r, dst_r, zeros_row)

    h0, stats1 = pl.pallas_call(
        _tc_b1_body,
        grid=(NBLK,),
        in_specs=[pl.BlockSpec((2, BLK, DH), stk),
                  pl.BlockSpec((2, BLK, DH), stk),
                  col,
                  pl.BlockSpec((1, d1), lambda i: (0, 0))],
        out_specs=(pl.BlockSpec((BLK, d1), row),
                   pl.BlockSpec((2, d1), lambda i: (0, 0))),
        out_shape=(jax.ShapeDtypeStruct((N, d1), jnp.float32),
                   jax.ShapeDtypeStruct((2, d1), jnp.float32)),
    )(acc0, ys0, dis, b0.reshape(1, d1))

    ys1 = pl.pallas_call(
        _tc_b2_body,
        grid=(NBLK,),
        in_specs=[pl.BlockSpec((BLK, d1), row),
                  col,
                  pl.BlockSpec((2, d1), lambda i: (0, 0)),
                  pl.BlockSpec((1, d1), lambda i: (0, 0)),
                  pl.BlockSpec((1, d1), lambda i: (0, 0)),
                  pl.BlockSpec((d1, d2), lambda i: (0, 0))],
        out_specs=pl.BlockSpec((2, BLK, DH), stk),
        out_shape=jax.ShapeDtypeStruct((2, N, DH), jnp.float32),
    )(h0, dis, stats1, bn1_g.reshape(1, d1), bn1_b.reshape(1, d1), W1)

    acc1 = _make_agg_kernel()(ys1, src_r, dst_r, zeros_row)

    _, out = pl.pallas_call(
        _tc_c_body,
        grid=(NBLK,),
        in_specs=[pl.BlockSpec((2, BLK, DH), stk),
                  pl.BlockSpec((2, BLK, DH), stk),
                  col,
                  pl.BlockSpec((1, d2), lambda i: (0, 0)),
                  pl.BlockSpec((BLK, 1), row),
                  pl.BlockSpec((d2, d2 // 4), lambda i: (0, 0)),
                  pl.BlockSpec((1, d2 // 4), lambda i: (0, 0)),
                  pl.BlockSpec((d2 // 4, 10), lambda i: (0, 0)),
                  pl.BlockSpec((1, 10), lambda i: (0, 0))],
        out_specs=(pl.BlockSpec((16, d2 + 1), lambda i: (0, 0)),
                   pl.BlockSpec((16, 10), lambda i: (0, 0))),
        out_shape=(jax.ShapeDtypeStruct((16, d2 + 1), jnp.float32),
                   jax.ShapeDtypeStruct((16, 10), jnp.float32)),
    )(acc1, ys1, dis, b1.reshape(1, d2), batch.reshape(N, 1),
      l1W, l1b.reshape(1, d2 // 4), l2W, l2b.reshape(1, 10))

    return out
